# baseline (speedup 1.0000x reference)
import functools

import jax
import jax.numpy as jnp
from jax.experimental import pallas as pl
from jax.experimental.pallas import tpu as pltpu


def _hm_idx(t):
    return (t + 1) // 2


def _out_i(t, nb0):
    return jnp.where(t < 6, jnp.maximum((t - 2) // 2, 0),
                     jnp.where(t < nb0 + 4, t - 4, t - (nb0 + 4)))


def _out_jh(t, nb0):
    return jnp.where(t < nb0 + 4, 0, 1)


def _fused_body(n_agents, action_dim, bb_rows, ti, tj, nb0,
                actions_ref, states_ref, ww_ref, bw_ref, wb_ref, bb_ref,
                out_ref, mixed_s, b_s):
    N, A = n_agents, action_dim
    t = pl.program_id(0)
    is_hm = (t == 0) | (t == 1) | (t == 3) | (t == 5)

    @pl.when(is_hm)
    def _hypermix():
        states = states_ref[...]
        hw = jnp.dot(states, ww_ref[...],
                     preferred_element_type=jnp.float32) + bw_ref[...]
        b = jnp.dot(states, wb_ref[...],
                    preferred_element_type=jnp.float32) + bb_ref[...]
        acts = actions_ref[...]
        mixed = acts[:, 0:1] * jnp.abs(hw[:, 0:A])
        for n in range(1, N):
            mixed = mixed + acts[:, n:n + 1] * jnp.abs(hw[:, n * A:(n + 1) * A])
        row0 = _hm_idx(t) * bb_rows
        mixed_s[pl.ds(row0, bb_rows), :] = mixed
        b_s[pl.ds(row0, bb_rows), :] = b

    @pl.when(~is_hm)
    def _broadcast():
        rows = b_s[pl.ds(_out_i(t, nb0) * ti, ti), :]
        mix = mixed_s[pl.ds(_out_jh(t, nb0) * tj, tj), :]
        out_ref[...] = rows[:, None, :] + mix[None, :, :]


def kernel(actions, states, ww, bw, wb, bb):
    f32 = jnp.float32
    actions = jnp.asarray(actions, f32)
    states = jnp.asarray(states, f32)
    B, N = actions.shape
    S = states.shape[1]
    NA = ww.shape[1]
    A = wb.shape[1]
    assert NA == N * A
    assert B % 64 == 0

    BB = B // 4
    TI = 16
    TJ = B // 2
    NB0 = B // TI
    grid_len = 4 + 2 * NB0

    out = pl.pallas_call(
        functools.partial(_fused_body, N, A, BB, TI, TJ, NB0),
        grid=(grid_len,),
        in_specs=[
            pl.BlockSpec((BB, N), lambda t: (jnp.minimum(_hm_idx(t), 3), 0)),
            pl.BlockSpec((BB, S), lambda t: (jnp.minimum(_hm_idx(t), 3), 0)),
            pl.BlockSpec((S, NA), lambda t: (0, 0)),
            pl.BlockSpec((1, NA), lambda t: (0, 0)),
            pl.BlockSpec((S, A), lambda t: (0, 0)),
            pl.BlockSpec((1, A), lambda t: (0, 0)),
        ],
        out_specs=pl.BlockSpec(
            (TI, TJ, A),
            lambda t: (_out_i(t, NB0), _out_jh(t, NB0), 0)),
        out_shape=jax.ShapeDtypeStruct((B, B, A), f32),
        scratch_shapes=[pltpu.VMEM((B, A), f32),
                        pltpu.VMEM((B, A), f32)],
        compiler_params=pltpu.CompilerParams(
            dimension_semantics=("arbitrary",)),
    )(actions, states, ww.astype(f32), bw.astype(f32),
      wb.astype(f32), bb.astype(f32))
    return out


# speedup vs baseline: 1.0252x; 1.0252x over previous
import functools

import jax
import jax.numpy as jnp
from jax.experimental import pallas as pl
from jax.experimental.pallas import tpu as pltpu


def _fused_body(n_agents, action_dim, bb_rows, ti, n_hm,
                actions_ref, states_ref, ww_ref, bw_ref, wb_ref, bb_ref,
                out_ref, hw_s, mixed_s, b_s):
    N, A = n_agents, action_dim
    t = pl.program_id(0)
    r = t % 4
    row0 = r * bb_rows

    @pl.when(t < n_hm)
    def _hypermix():
        states = states_ref[...]
        hw = jnp.dot(states, ww_ref[...],
                     preferred_element_type=jnp.float32)
        b = jnp.dot(states, wb_ref[...],
                    preferred_element_type=jnp.float32)

        @pl.when(t < 4)
        def _first_half():
            hw_s[pl.ds(row0, bb_rows), :] = hw
            b_s[pl.ds(row0, bb_rows), :] = b

        @pl.when(t >= 4)
        def _second_half():
            hw_full = hw_s[pl.ds(row0, bb_rows), :] + hw + bw_ref[...]
            b_s[pl.ds(row0, bb_rows), :] += b + bb_ref[...]
            acts = actions_ref[...]
            mixed = acts[:, 0:1] * jnp.abs(hw_full[:, 0:A])
            for n in range(1, N):
                mixed = mixed + acts[:, n:n + 1] * jnp.abs(
                    hw_full[:, n * A:(n + 1) * A])
            mixed_s[pl.ds(row0, bb_rows), :] = mixed

    @pl.when(t >= n_hm)
    def _broadcast():
        rows = b_s[pl.ds((t - n_hm) * ti, ti), :]
        out_ref[...] = rows[:, None, :] + mixed_s[...][None, :, :]


def kernel(actions, states, ww, bw, wb, bb):
    f32 = jnp.float32
    actions = jnp.asarray(actions, f32)
    states = jnp.asarray(states, f32)
    B, N = actions.shape
    S = states.shape[1]
    NA = ww.shape[1]
    A = wb.shape[1]
    assert NA == N * A
    assert B % 64 == 0 and S % 2 == 0

    BB = B // 4
    SH = S // 2
    TI = 16
    N_HM = 8
    GI = B // TI

    out = pl.pallas_call(
        functools.partial(_fused_body, N, A, BB, TI, N_HM),
        grid=(N_HM + GI,),
        in_specs=[
            pl.BlockSpec((BB, N),
                         lambda t: (jnp.where(t < 8, t % 4, 3), 0)),
            pl.BlockSpec((BB, SH),
                         lambda t: (jnp.where(t < 8, t % 4, 3),
                                    jnp.where(t < 8, t // 4, 1))),
            pl.BlockSpec((SH, NA),
                         lambda t: (jnp.where(t < 8, t // 4, 1), 0)),
            pl.BlockSpec((1, NA), lambda t: (0, 0)),
            pl.BlockSpec((SH, A),
                         lambda t: (jnp.where(t < 8, t // 4, 1), 0)),
            pl.BlockSpec((1, A), lambda t: (0, 0)),
        ],
        out_specs=pl.BlockSpec(
            (TI, B, A), lambda t: (jnp.maximum(t - 8, 0), 0, 0)),
        out_shape=jax.ShapeDtypeStruct((B, B, A), f32),
        scratch_shapes=[pltpu.VMEM((B, NA), f32),
                        pltpu.VMEM((B, A), f32),
                        pltpu.VMEM((B, A), f32)],
        compiler_params=pltpu.CompilerParams(
            dimension_semantics=("arbitrary",)),
    )(actions, states, ww.astype(f32), bw.astype(f32),
      wb.astype(f32), bb.astype(f32))
    return out


# speedup vs baseline: 1.0419x; 1.0164x over previous
import functools

import jax
import jax.numpy as jnp
from jax.experimental import pallas as pl
from jax.experimental.pallas import tpu as pltpu


def _fused_body(n_agents, action_dim, gb, bb_rows, ti,
                actions_ref, states_ref, ww_ref, bw_ref, wb_ref, bb_ref,
                out_ref, mixed_s, b_s):
    N, A = n_agents, action_dim
    t = pl.program_id(0)

    @pl.when(t < gb)
    def _hypermix():
        states = states_ref[...]
        hw = jnp.dot(states, ww_ref[...],
                     preferred_element_type=jnp.float32) + bw_ref[...]
        b = jnp.dot(states, wb_ref[...],
                    preferred_element_type=jnp.float32) + bb_ref[...]
        acts = actions_ref[...]
        mixed = acts[:, 0:1] * jnp.abs(hw[:, 0:A])
        for n in range(1, N):
            mixed = mixed + acts[:, n:n + 1] * jnp.abs(hw[:, n * A:(n + 1) * A])
        row0 = t * bb_rows
        mixed_s[pl.ds(row0, bb_rows), :] = mixed
        b_s[pl.ds(row0, bb_rows), :] = b

    @pl.when(t >= gb)
    def _broadcast():
        rows = b_s[pl.ds((t - gb) * ti, ti), :]
        out_ref[...] = rows[:, None, :] + mixed_s[...][None, :, :]


def kernel(actions, states, ww, bw, wb, bb):
    f32 = jnp.float32
    actions = jnp.asarray(actions, f32)
    states = jnp.asarray(states, f32)
    B, N = actions.shape
    S = states.shape[1]
    NA = ww.shape[1]
    A = wb.shape[1]
    assert NA == N * A
    assert B % 64 == 0

    BB = B // 4
    TI = 16
    GB, GI = B // BB, B // TI

    out = pl.pallas_call(
        functools.partial(_fused_body, N, A, GB, BB, TI),
        grid=(GB + GI,),
        in_specs=[
            pl.BlockSpec((BB, N), lambda t: (jnp.minimum(t, 3), 0)),
            pl.BlockSpec((BB, S), lambda t: (jnp.minimum(t, 3), 0)),
            pl.BlockSpec((S, NA), lambda t: (0, 0)),
            pl.BlockSpec((1, NA), lambda t: (0, 0)),
            pl.BlockSpec((S, A), lambda t: (0, 0)),
            pl.BlockSpec((1, A), lambda t: (0, 0)),
        ],
        out_specs=pl.BlockSpec(
            (TI, B, A), lambda t: (jnp.maximum(t - GB, 0), 0, 0)),
        out_shape=jax.ShapeDtypeStruct((B, B, A), f32),
        scratch_shapes=[pltpu.VMEM((B, A), f32),
                        pltpu.VMEM((B, A), f32)],
        compiler_params=pltpu.CompilerParams(
            dimension_semantics=("arbitrary",)),
    )(actions, states, ww.astype(f32), bw.astype(f32),
      wb.astype(f32), bb.astype(f32))
    return out
